# baseline (speedup 1.0000x reference)
import functools

import jax
import jax.numpy as jnp
from jax.experimental import pallas as pl
from jax.experimental.pallas import tpu as pltpu


def _matmul_kernel(a_ref, f_ref, o_ref):
    o_ref[...] = jnp.dot(
        a_ref[...],
        f_ref[...],
        preferred_element_type=jnp.float32,
        precision=jax.lax.Precision.DEFAULT,
    )


@functools.partial(jax.jit, static_argnames=("bm",))
def _matmul(adj, features, bm: int):
    m, k = adj.shape
    _, n = features.shape
    return pl.pallas_call(
        _matmul_kernel,
        grid=(pl.cdiv(m, bm),),
        in_specs=[
            pl.BlockSpec((bm, k), lambda i: (i, 0)),
            pl.BlockSpec((k, n), lambda i: (0, 0)),
        ],
        out_specs=pl.BlockSpec((bm, n), lambda i: (i, 0)),
        out_shape=jax.ShapeDtypeStruct((m, n), jnp.float32),
        compiler_params=pltpu.CompilerParams(
            dimension_semantics=("parallel",)
        ),
    )(adj, features)


def kernel(features, adj):
    return _matmul(adj, features, bm=320)


# speedup vs baseline: 1.0016x; 1.0016x over previous
import functools

import jax
import jax.numpy as jnp
from jax.experimental import pallas as pl
from jax.experimental.pallas import tpu as pltpu


def _matmul_kernel(a_ref, f_ref, o_ref):
    o_ref[...] = jnp.dot(
        a_ref[...],
        f_ref[...],
        preferred_element_type=jnp.float32,
        precision=jax.lax.Precision.DEFAULT,
    )


@functools.partial(jax.jit, static_argnames=("bm",))
def _matmul(adj, features, bm: int):
    m, k = adj.shape
    _, n = features.shape
    return pl.pallas_call(
        _matmul_kernel,
        grid=(pl.cdiv(m, bm),),
        in_specs=[
            pl.BlockSpec((bm, k), lambda i: (i, 0)),
            pl.BlockSpec((k, n), lambda i: (0, 0)),
        ],
        out_specs=pl.BlockSpec((bm, n), lambda i: (i, 0)),
        out_shape=jax.ShapeDtypeStruct((m, n), jnp.float32),
        compiler_params=pltpu.CompilerParams(
            dimension_semantics=("parallel",)
        ),
    )(adj, features)


def kernel(features, adj):
    return _matmul(adj, features, bm=400)
